# Initial kernel scaffold; baseline (speedup 1.0000x reference)
import jax
import jax.numpy as jnp
from jax.experimental import pallas as pl


def kernel(x, n, W_rank, b_rank, W1, b1, g1, be1, W2, b2, V1, vb1, gv, bv, V2, vb2, W_card, b_card):
    raise NotImplementedError("write your pallas kernel here")


# speedup vs baseline: 1.3411x; 1.3411x over previous
import functools

import jax
import jax.numpy as jnp
from jax import lax
from jax.experimental import pallas as pl
from jax.experimental.pallas import tpu as pltpu
from jax.experimental.pallas import tpu_sc as plsc

_B = 512
_D = 64
_H = 128
_MAXN = 512
_TOTAL = _B * (_B - 1) // 2

_TOK_BLK = 256
_N_BLK = _TOTAL // _TOK_BLK

_CHUNK = 128
_NCHUNK = _TOTAL // _CHUNK
_NWORKER = 32


def _mish(v):
    return v * jnp.tanh(jnp.logaddexp(v, 0.0))


def _ln(v, g, b):
    m = jnp.mean(v, axis=-1, keepdims=True)
    var = jnp.var(v, axis=-1, keepdims=True)
    return (v - m) / jnp.sqrt(var + 1e-5) * g + b


def _feat_body(x_ref, wrt_ref, br_ref, v1t_ref, vb1_ref, gv_ref, bv_ref,
               v2t_ref, vb2_ref, mag_ref, mx_ref, vf_ref):
    xb = x_ref[...]
    m = jnp.dot(xb, wrt_ref[...]) + br_ref[0, 0]
    mag_ref[...] = m
    bm = jnp.max(m, axis=0, keepdims=True)
    i = pl.program_id(0)

    @pl.when(i == 0)
    def _():
        mx_ref[...] = bm

    @pl.when(i > 0)
    def _():
        mx_ref[...] = jnp.maximum(mx_ref[...], bm)

    h = jnp.dot(xb, v1t_ref[...]) + vb1_ref[...]
    h = _mish(_ln(h, gv_ref[...], bv_ref[...]))
    vf_ref[...] = jnp.dot(h, v2t_ref[...]) + vb2_ref[...]


def _features(x, W_rank, b_rank, V1, vb1, gv, bv, V2, vb2):
    full = lambda i: (0, 0)
    return pl.pallas_call(
        _feat_body,
        grid=(_N_BLK,),
        in_specs=[
            pl.BlockSpec((_TOK_BLK, _D), lambda i: (i, 0)),
            pl.BlockSpec((_D, 1), full),
            pl.BlockSpec((1, 1), full),
            pl.BlockSpec((_D, 96), full),
            pl.BlockSpec((1, 96), full),
            pl.BlockSpec((1, 96), full),
            pl.BlockSpec((1, 96), full),
            pl.BlockSpec((96, _H), full),
            pl.BlockSpec((1, _H), full),
        ],
        out_specs=[
            pl.BlockSpec((_TOK_BLK, 1), lambda i: (i, 0)),
            pl.BlockSpec((1, 1), full),
            pl.BlockSpec((_TOK_BLK, _H), lambda i: (i, 0)),
        ],
        out_shape=[
            jax.ShapeDtypeStruct((_TOTAL, 1), jnp.float32),
            jax.ShapeDtypeStruct((1, 1), jnp.float32),
            jax.ShapeDtypeStruct((_TOTAL, _H), jnp.float32),
        ],
    )(x, W_rank.T, b_rank.reshape(1, 1), V1.T, vb1.reshape(1, -1),
      gv.reshape(1, -1), bv.reshape(1, -1), V2.T, vb2.reshape(1, -1))


def _kf_body(w1t_ref, b1_ref, g1_ref, be1_ref, w2t_ref, b2_ref, out_ref):
    h = w1t_ref[...] + b1_ref[...]
    h = _mish(_ln(h, g1_ref[...], be1_ref[...]))
    out_ref[...] = jnp.dot(h, w2t_ref[...]) + b2_ref[...]


def _kf_table(W1, b1, g1, be1, W2, b2):
    return pl.pallas_call(
        _kf_body,
        out_shape=jax.ShapeDtypeStruct((_MAXN, _H), jnp.float32),
    )(W1.T, b1.reshape(1, -1), g1.reshape(1, -1), be1.reshape(1, -1),
      W2.T, b2.reshape(1, -1))


def _sc_body(vf_hbm, kf_hbm, perm_hbm, key_hbm, batch_hbm, out_hbm,
             pidx, kidx, bidx, vrows, krows, zstage, zacc, sem1, sem2):
    c = lax.axis_index("c")
    s = lax.axis_index("s")
    wid = s * 2 + c

    def _zrow(r, _):
        for j in range(8):
            zstage[r, pl.ds(j * 16, 16)] = jnp.zeros((16,), jnp.float32)
        return 0

    lax.fori_loop(0, 64, _zrow, 0)

    @pl.when(s == 0)
    def _():
        for k in range(8):
            pltpu.sync_copy(zstage, zacc.at[pl.ds(k * 64, 64)])

    plsc.subcore_barrier()

    nw = (_NCHUNK - wid + _NWORKER - 1) // _NWORKER

    def _chunk(t, _):
        base = (wid + t * _NWORKER) * _CHUNK
        pltpu.sync_copy(perm_hbm.at[pl.ds(base, _CHUNK)], pidx)
        pltpu.sync_copy(key_hbm.at[pl.ds(base, _CHUNK)], kidx)
        pltpu.sync_copy(batch_hbm.at[pl.ds(base, _CHUNK)], bidx)
        cp1 = pltpu.async_copy(vf_hbm.at[pidx], vrows, sem1)
        cp2 = pltpu.async_copy(kf_hbm.at[kidx], krows, sem2)
        cp1.wait()
        cp2.wait()

        def _mul(r, _):
            for j in range(8):
                sl = pl.ds(j * 16, 16)
                vrows[r, sl] = vrows[r, sl] * krows[r, sl]
            return 0

        lax.fori_loop(0, _CHUNK, _mul, 0)
        pltpu.sync_copy(vrows, zacc.at[bidx], add=True)
        return 0

    lax.fori_loop(0, nw, _chunk, 0)
    plsc.subcore_barrier()
    rows = _B // 16
    pltpu.sync_copy(zacc.at[pl.ds(s * rows, rows)],
                    out_hbm.at[c, pl.ds(s * rows, rows)])


def _sc_combine(vf, kf, perm, keys, batch):
    mesh = plsc.VectorSubcoreMesh(core_axis_name="c", subcore_axis_name="s")
    f = functools.partial(
        pl.kernel,
        out_type=jax.ShapeDtypeStruct((2, _B, _H), jnp.float32),
        mesh=mesh,
        scratch_types=[
            pltpu.VMEM((_CHUNK,), jnp.int32),
            pltpu.VMEM((_CHUNK,), jnp.int32),
            pltpu.VMEM((_CHUNK,), jnp.int32),
            pltpu.VMEM((_CHUNK, _H), jnp.float32),
            pltpu.VMEM((_CHUNK, _H), jnp.float32),
            pltpu.VMEM((64, _H), jnp.float32),
            pltpu.VMEM_SHARED((_B, _H), jnp.float32),
            pltpu.SemaphoreType.DMA,
            pltpu.SemaphoreType.DMA,
        ],
    )(_sc_body)
    return f(vf, kf, perm, keys, batch)


def _epi_body(z0_ref, z1_ref, nf_ref, wc_ref, bc_ref, out_ref):
    out_ref[...] = (z0_ref[...] + z1_ref[...]
                    + nf_ref[...] * wc_ref[...] + bc_ref[...])


def _epilogue(zp, n, W_card, b_card):
    return pl.pallas_call(
        _epi_body,
        out_shape=jax.ShapeDtypeStruct((_B, _H), jnp.float32),
    )(zp[0], zp[1], n.astype(jnp.float32).reshape(_B, 1),
      W_card.reshape(1, _H), b_card.reshape(1, _H))


def kernel(x, n, W_rank, b_rank, W1, b1, g1, be1, W2, b2,
           V1, vb1, gv, bv, V2, vb2, W_card, b_card):
    total = x.shape[0]
    nb = n.shape[0]

    mag, mx, vf = _features(x, W_rank, b_rank, V1, vb1, gv, bv, V2, vb2)
    kf = _kf_table(W1, b1, g1, be1, W2, b2)

    batch = jnp.repeat(jnp.arange(nb), n, total_repeat_length=total)
    csum = jnp.cumsum(n)
    offsets = csum - n
    keys = (jnp.arange(total)
            - jnp.repeat(offsets, n, total_repeat_length=total)).astype(jnp.int32)

    max_mag = mx[0, 0] + 0.0001
    new_mag = mag[:, 0] + batch.astype(x.dtype) * max_mag
    perm = jnp.argsort(new_mag).astype(jnp.int32)

    zp = _sc_combine(vf, kf, perm, keys, batch.astype(jnp.int32))
    return _epilogue(zp, n, W_card, b_card)
